# baseline (speedup 1.0000x reference)
import functools
import jax
import jax.numpy as jnp
from jax import lax
from jax.experimental import pallas as pl
from jax.experimental.pallas import tpu as pltpu
from jax.experimental.pallas import tpu_sc as plsc

_DIM = 3
_SPATIAL = 32
_BATCH = 8
_N = 500000
_C = 128
_G = _BATCH * _SPATIAL ** 3

_NS = 16
_BLK = 512
_NFULL = _N // _BLK
_TAIL = _N - _NFULL * _BLK
_SLOTS = _NFULL // _NS
_NDUM = 128
_REAL = 7552
_NPASS = -(-_G // _REAL)
_SC0_PASSES = (_NPASS + 1) // 2

_mesh = plsc.VectorSubcoreMesh(core_axis_name="c", subcore_axis_name="s")


@functools.partial(
    pl.kernel,
    mesh=_mesh,
    out_type=jax.ShapeDtypeStruct((_G, _C), jnp.float32),
    scratch_types=[
        pltpu.VMEM((_BLK, _C), jnp.float32),
        pltpu.VMEM((_BLK,), jnp.int32),
        pltpu.VMEM((_BLK,), jnp.int32),
        pltpu.VMEM((_BLK,), jnp.int32),
        pltpu.VMEM((_BLK,), jnp.int32),
        pltpu.VMEM((_BLK,), jnp.float32),
        pltpu.VMEM((_BLK,), jnp.int32),
        pltpu.VMEM_SHARED((_NDUM + _REAL, _C), jnp.float32),
    ],
)
def _revoxel(x_hbm, y_hbm, z_hbm, b_hbm, f1_hbm, feat_hbm, zero_hbm, out_hbm,
             ftile, xtile, ytile, ztile, btile, f1tile, itile, chunk):
    cid = lax.axis_index("c")
    sid = lax.axis_index("s")
    lanes = lax.iota(jnp.int32, 16)

    def compute_indices(nvec, lo, hi):
        def vec_body(i, _):
            base = i * 16
            sl = pl.ds(base, 16)
            x = xtile[sl]
            y = ytile[sl]
            z = ztile[sl]
            bt = btile[sl]
            f1 = f1tile[sl]
            lin = (bt * (_SPATIAL ** 3) + x * (_SPATIAL ** 2)
                   + y * _SPATIAL + z)
            ok = (f1 > 0.5) & (lin >= lo) & (lin < hi)
            idx = jnp.where(ok, lin - lo + _NDUM, lanes)
            itile[sl] = idx
            return _
        lax.fori_loop(0, nvec, vec_body, None)
        def pad_body(i, _):
            itile[pl.ds(i * 16, 16)] = lanes
            return _
        lax.fori_loop(nvec, _BLK // 16, pad_body, None)

    def load_block(base, n):
        pltpu.sync_copy(x_hbm.at[pl.ds(base, n)], xtile.at[pl.ds(0, n)])
        pltpu.sync_copy(y_hbm.at[pl.ds(base, n)], ytile.at[pl.ds(0, n)])
        pltpu.sync_copy(z_hbm.at[pl.ds(base, n)], ztile.at[pl.ds(0, n)])
        pltpu.sync_copy(b_hbm.at[pl.ds(base, n)], btile.at[pl.ds(0, n)])
        pltpu.sync_copy(f1_hbm.at[pl.ds(base, n)], f1tile.at[pl.ds(0, n)])
        pltpu.sync_copy(feat_hbm.at[pl.ds(base, n)], ftile.at[pl.ds(0, n)])

    def do_pass(p):
        lo = p * _REAL
        rows = min(_G - lo, _REAL)
        quota = (rows + _NDUM) // _NS
        dquota = rows // _NS

        pltpu.sync_copy(zero_hbm.at[pl.ds(0, quota)],
                        chunk.at[pl.ds(sid * quota, quota)])
        plsc.subcore_barrier()

        def block_body(j, _):
            b = j * _NS + sid
            load_block(b * _BLK, _BLK)
            compute_indices(_BLK // 16, lo, lo + rows)
            pltpu.sync_copy(ftile, chunk.at[itile], add=True)
            return _
        lax.fori_loop(0, _SLOTS, block_body, None)

        @pl.when(sid == 0)
        def _():
            load_block(_NFULL * _BLK, _TAIL)
            compute_indices(_TAIL // 16, lo, lo + rows)
            pltpu.sync_copy(ftile, chunk.at[itile], add=True)

        plsc.subcore_barrier()
        pltpu.sync_copy(chunk.at[pl.ds(_NDUM + sid * dquota, dquota)],
                        out_hbm.at[pl.ds(lo + sid * dquota, dquota)])
        plsc.subcore_barrier()

    for p in range(_NPASS):
        owner = 0 if p < _SC0_PASSES else 1

        @pl.when(cid == owner)
        def _(p=p):
            do_pass(p)


def kernel(spatial_locations, features, batch_idx):
    sp = spatial_locations.astype(jnp.int32)
    x = sp[:, 0]
    y = sp[:, 1]
    z = sp[:, 2]
    f1 = features[:, 1]
    zeros = jnp.zeros(((_REAL + _NDUM) // _NS, _C), dtype=jnp.float32)
    return _revoxel(x, y, z, batch_idx.astype(jnp.int32), f1, features, zeros)


# speedup vs baseline: 1.2620x; 1.2620x over previous
import functools
import jax
import jax.numpy as jnp
from jax import lax
from jax.experimental import pallas as pl
from jax.experimental.pallas import tpu as pltpu
from jax.experimental.pallas import tpu_sc as plsc

_SPATIAL = 32
_BATCH = 8
_N = 500000
_C = 128
_G = _BATCH * _SPATIAL ** 3

_NC = 2
_NS = 16
_NW = _NC * _NS
_BLK = 512
_NFULL = _N // _BLK
_TAIL = _N - _NFULL * _BLK
_SLOTS = _NFULL // _NS
_WSLOTS = -(-_NFULL // _NW)
_NDUM = 128
_REAL = 7552
_NPASS = -(-_G // _REAL)
_SC0_PASSES = (_NPASS + 1) // 2

_mesh = plsc.VectorSubcoreMesh(core_axis_name="c", subcore_axis_name="s")


@functools.partial(
    pl.kernel,
    mesh=_mesh,
    out_type=jax.ShapeDtypeStruct((_N,), jnp.int32),
    scratch_types=[
        pltpu.VMEM((_BLK,), jnp.int32),
        pltpu.VMEM((_BLK,), jnp.int32),
        pltpu.VMEM((_BLK,), jnp.int32),
        pltpu.VMEM((_BLK,), jnp.int32),
        pltpu.VMEM((_BLK,), jnp.float32),
        pltpu.VMEM((_BLK,), jnp.int32),
    ],
)
def _linindex(x_hbm, y_hbm, z_hbm, b_hbm, f1_hbm, out_hbm,
              xtile, ytile, ztile, btile, f1tile, ltile):
    wid = lax.axis_index("s") * _NC + lax.axis_index("c")

    def do_block(base, n):
        sl0 = pl.ds(0, n)
        pltpu.sync_copy(x_hbm.at[pl.ds(base, n)], xtile.at[sl0])
        pltpu.sync_copy(y_hbm.at[pl.ds(base, n)], ytile.at[sl0])
        pltpu.sync_copy(z_hbm.at[pl.ds(base, n)], ztile.at[sl0])
        pltpu.sync_copy(b_hbm.at[pl.ds(base, n)], btile.at[sl0])
        pltpu.sync_copy(f1_hbm.at[pl.ds(base, n)], f1tile.at[sl0])
        def vec_body(i, _):
            sl = pl.ds(i * 16, 16)
            lin = (btile[sl] * (_SPATIAL ** 3) + xtile[sl] * (_SPATIAL ** 2)
                   + ytile[sl] * _SPATIAL + ztile[sl])
            ltile[sl] = jnp.where(f1tile[sl] > 0.5, lin, -1)
            return _
        lax.fori_loop(0, n // 16, vec_body, None)
        pltpu.sync_copy(ltile.at[sl0], out_hbm.at[pl.ds(base, n)])

    def block_body(j, _):
        b = j * _NW + wid
        @pl.when(b < _NFULL)
        def _():
            do_block(b * _BLK, _BLK)
        return _
    lax.fori_loop(0, _WSLOTS, block_body, None)

    @pl.when(wid == 0)
    def _():
        do_block(_NFULL * _BLK, _TAIL)


@functools.partial(
    pl.kernel,
    mesh=_mesh,
    out_type=jax.ShapeDtypeStruct((_G, _C), jnp.float32),
    scratch_types=[
        pltpu.VMEM((_BLK, _C), jnp.float32),
        pltpu.VMEM((_BLK,), jnp.int32),
        pltpu.VMEM_SHARED((_NDUM + _REAL, _C), jnp.float32),
    ],
)
def _revoxel(lin_hbm, feat_hbm, zero_hbm, out_hbm, ftile, itile, chunk):
    cid = lax.axis_index("c")
    sid = lax.axis_index("s")
    lanes = lax.iota(jnp.int32, 16)

    def scatter_block(base, n, lo, hi):
        pltpu.sync_copy(lin_hbm.at[pl.ds(base, n)], itile.at[pl.ds(0, n)])
        pltpu.sync_copy(feat_hbm.at[pl.ds(base, n)], ftile.at[pl.ds(0, n)])
        def vec_body(i, _):
            sl = pl.ds(i * 16, 16)
            lin = itile[sl]
            ok = (lin >= lo) & (lin < hi)
            itile[sl] = jnp.where(ok, lin - lo + _NDUM, lanes)
            return _
        lax.fori_loop(0, n // 16, vec_body, None)
        def pad_body(i, _):
            itile[pl.ds(i * 16, 16)] = lanes
            return _
        lax.fori_loop(n // 16, _BLK // 16, pad_body, None)
        pltpu.sync_copy(ftile, chunk.at[itile], add=True)

    def do_pass(p):
        lo = p * _REAL
        rows = min(_G - lo, _REAL)
        quota = (rows + _NDUM) // _NS
        dquota = rows // _NS

        pltpu.sync_copy(zero_hbm.at[pl.ds(0, quota)],
                        chunk.at[pl.ds(sid * quota, quota)])
        plsc.subcore_barrier()

        def block_body(j, _):
            b = j * _NS + sid
            scatter_block(b * _BLK, _BLK, lo, lo + rows)
            return _
        lax.fori_loop(0, _SLOTS, block_body, None)

        @pl.when(sid == 0)
        def _():
            scatter_block(_NFULL * _BLK, _TAIL, lo, lo + rows)

        plsc.subcore_barrier()
        pltpu.sync_copy(chunk.at[pl.ds(_NDUM + sid * dquota, dquota)],
                        out_hbm.at[pl.ds(lo + sid * dquota, dquota)])
        plsc.subcore_barrier()

    for p in range(_NPASS):
        owner = 0 if p < _SC0_PASSES else 1

        @pl.when(cid == owner)
        def _(p=p):
            do_pass(p)


def kernel(spatial_locations, features, batch_idx):
    sp = spatial_locations.astype(jnp.int32)
    lin = _linindex(sp[:, 0], sp[:, 1], sp[:, 2],
                    batch_idx.astype(jnp.int32), features[:, 1])
    zeros = jnp.zeros(((_REAL + _NDUM) // _NS, _C), dtype=jnp.float32)
    return _revoxel(lin, features, zeros)
